# baseline (speedup 1.0000x reference)
import functools

import jax
import jax.numpy as jnp
from jax import lax
from jax.experimental import pallas as pl
from jax.experimental.pallas import tpu as pltpu
from jax.experimental.pallas import tpu_sc as plsc

NUM_CLASS = 100000
DATASET_NUM = 8
CTX_DIM = 512
N_CLS_CTX = 4
N_DM_CTX = 1
B = 1024
SEQ = 77

NC = 2
NS = 16
NW = NC * NS
BPW = B // NW

_mesh = plsc.VectorSubcoreMesh(core_axis_name="c", subcore_axis_name="s")


@functools.partial(
    pl.kernel,
    out_type=jax.ShapeDtypeStruct((B, SEQ, CTX_DIM), jnp.float32),
    mesh=_mesh,
    compiler_params=pltpu.CompilerParams(use_tc_tiling_on_sc=False),
    scratch_types=[
        pltpu.VMEM((BPW,), jnp.int32),
        pltpu.VMEM((BPW,), jnp.int32),
        pltpu.VMEM((BPW, N_CLS_CTX, CTX_DIM), jnp.float32),
        pltpu.VMEM((BPW, N_DM_CTX, CTX_DIM), jnp.float32),
        pltpu.VMEM((5, CTX_DIM), jnp.float32),
        pltpu.VMEM((2, CTX_DIM), jnp.float32),
        pltpu.VMEM((65, CTX_DIM), jnp.float32),
        pltpu.SemaphoreType.DMA,
        pltpu.SemaphoreType.DMA,
    ],
)
def _assemble(label_h, domain_h, cls_h, dm_h, pref_h, inter_h, suf_h, out_h,
              idx_v, didx_v, rows_v, drows_v, pref_v, inter_v, suf_v,
              gsem, wsem):
    wid = lax.axis_index("s") * NC + lax.axis_index("c")
    base = wid * BPW

    pltpu.sync_copy(label_h.at[pl.ds(base, BPW)], idx_v)
    pltpu.sync_copy(domain_h.at[pl.ds(base, BPW)], didx_v)
    gcls = pltpu.async_copy(cls_h.at[idx_v], rows_v, gsem)
    gdom = pltpu.async_copy(dm_h.at[didx_v], drows_v, gsem)
    pltpu.sync_copy(pref_h.at[0], pref_v)
    pltpu.sync_copy(inter_h.at[0], inter_v)
    pltpu.sync_copy(suf_h.at[0], suf_v)
    gcls.wait()
    gdom.wait()

    def start_row(i):
        b = base + i
        cps = (
            pltpu.async_copy(pref_v, out_h.at[b, pl.ds(0, 5)], wsem),
            pltpu.async_copy(rows_v.at[i], out_h.at[b, pl.ds(5, N_CLS_CTX)], wsem),
            pltpu.async_copy(inter_v, out_h.at[b, pl.ds(9, 2)], wsem),
            pltpu.async_copy(drows_v.at[i], out_h.at[b, pl.ds(11, N_DM_CTX)], wsem),
            pltpu.async_copy(suf_v, out_h.at[b, pl.ds(12, 65)], wsem),
        )
        return cps

    def wait_row(cps):
        for cp in cps:
            cp.wait()

    def body(i, _):
        cps = start_row(i)
        wait_row(cps)
        return 0

    lax.fori_loop(0, BPW, body, 0)


def kernel(label, domain, clsctx, dmctx, token_prefix_domain,
           token_intermediate_domain, token_suffix_domain):
    return _assemble(label.astype(jnp.int32), domain.astype(jnp.int32),
                     clsctx, dmctx, token_prefix_domain,
                     token_intermediate_domain, token_suffix_domain)
